# baseline (speedup 1.0000x reference)
import functools

import jax
import jax.numpy as jnp
from jax import lax
from jax.experimental import pallas as pl
from jax.experimental.pallas import tpu as pltpu
from jax.experimental.pallas import tpu_sc as plsc

_CHUNK = 128
_ROW = 12
_GRP = 48
_L = 16


def _sc_gather(idx, table48):
    batch = idx.shape[0]
    info = plsc.get_sparse_core_info()
    nw = info.num_cores * info.num_subcores
    bpw = batch // nw
    nchunk = max(1, bpw // _CHUNK)
    chunk = bpw // nchunk
    nreg = bpw * _ROW // _L
    mesh = plsc.VectorSubcoreMesh(core_axis_name="c", subcore_axis_name="s")

    @functools.partial(
        pl.kernel,
        out_type=jax.ShapeDtypeStruct((batch * _ROW,), jnp.float32),
        mesh=mesh,
        scratch_types=[
            pltpu.VMEM((bpw,), jnp.int32),
            pltpu.VMEM((bpw,), jnp.int32),
            pltpu.VMEM((bpw, _GRP), jnp.float32),
            pltpu.VMEM((bpw * _ROW,), jnp.float32),
            pltpu.SemaphoreType.DMA,
        ],
        compiler_params=pltpu.CompilerParams(
            use_tc_tiling_on_sc=False, needs_layout_passes=False),
    )
    def k(idx_hbm, table_hbm, out_hbm, idx_v, gidx_v, rows_v, out_v, sem):
        wid = lax.axis_index("s") * info.num_cores + lax.axis_index("c")
        base = wid * bpw
        pltpu.sync_copy(idx_hbm.at[pl.ds(base, bpw)], idx_v)
        for r in range(bpw // _L):
            gidx_v[pl.ds(r * _L, _L)] = idx_v[pl.ds(r * _L, _L)] >> 2
        copies = [
            pltpu.async_copy(
                table_hbm.at[gidx_v.at[pl.ds(c * chunk, chunk)]],
                rows_v.at[pl.ds(c * chunk, chunk)],
                sem,
            )
            for c in range(nchunk)
        ]
        for cp in copies:
            cp.wait()
        lane = lax.iota(jnp.int32, _L)
        bpat = [(j * _L) // _ROW
                + jnp.where(lane >= _ROW - (j * _L) % _ROW, 1, 0)
                for j in range(3)]
        tpat = [j * _L + lane - _ROW * bpat[j] for j in range(3)]
        ngroup = nreg // 3
        unroll = 8

        def body(it, carry):
            g0 = it * unroll
            for gg in range(unroll):
                g = g0 + gg
                for j in range(3):
                    b = g * 4 + bpat[j]
                    m = plsc.load_gather(idx_v, [b])
                    off = (m & 3) * _ROW + tpat[j]
                    vals = plsc.load_gather(rows_v, [b, off])
                    out_v[pl.ds(g * 3 * _L + j * _L, _L)] = vals
            return carry

        lax.fori_loop(0, ngroup // unroll, body, jnp.int32(0))
        pltpu.sync_copy(out_v, out_hbm.at[pl.ds(base * _ROW, bpw * _ROW)])

    return k(idx, table48)


def kernel(projection, cam_idx):
    n, r, c = projection.shape
    batch = cam_idx.shape[1]
    table48 = projection.reshape(n * r * c // _GRP, _GRP)
    out = _sc_gather(cam_idx[1], table48)
    return out.reshape(batch, r, c)


# speedup vs baseline: 4.8121x; 4.8121x over previous
import functools

import jax
import jax.numpy as jnp
from jax import lax
from jax.experimental import pallas as pl
from jax.experimental.pallas import tpu as pltpu
from jax.experimental.pallas import tpu_sc as plsc

_CHUNK = 128
_L = 16


def kernel(projection, cam_idx):
    n, r, c = projection.shape
    batch = cam_idx.shape[1]
    planes = r * c
    pv = n // _L
    info = plsc.get_sparse_core_info()
    nw = info.num_cores * info.num_subcores
    bpw = batch // nw
    nbt = bpw // _CHUNK
    ng = bpw * planes
    nchunk = ng // _CHUNK
    mesh = plsc.VectorSubcoreMesh(core_axis_name="c", subcore_axis_name="s")

    @functools.partial(
        pl.kernel,
        out_type=jax.ShapeDtypeStruct((r, batch // _CHUNK, c, _CHUNK),
                                      jnp.float32),
        mesh=mesh,
        scratch_types=[
            pltpu.VMEM((bpw,), jnp.int32),
            pltpu.VMEM((bpw,), jnp.int32),
            pltpu.VMEM((ng,), jnp.int32),
            pltpu.VMEM((ng, _L), jnp.float32),
            pltpu.VMEM((r, nbt, c, _CHUNK), jnp.float32),
            pltpu.SemaphoreType.DMA,
        ],
        compiler_params=pltpu.CompilerParams(
            use_tc_tiling_on_sc=False, needs_layout_passes=False),
    )
    def k(tab_hbm, cam_hbm, out_hbm, idx_v, offs_v, gidx_v, rows_v, out_v, sem):
        wid = lax.axis_index("s") * info.num_cores + lax.axis_index("c")
        base = wid * bpw
        pltpu.sync_copy(cam_hbm.at[1, pl.ds(base, bpw)], idx_v)

        def prep(rr, carry):
            v = idx_v[pl.ds(rr * _L, _L)]
            offs_v[pl.ds(rr * _L, _L)] = v & (_L - 1)
            g0 = v >> 4
            for kk in range(planes):
                gidx_v[pl.ds(kk * bpw + rr * _L, _L)] = g0 + kk * pv
            return carry

        lax.fori_loop(0, bpw // _L, prep, jnp.int32(0))

        copies = [
            pltpu.async_copy(
                tab_hbm.at[gidx_v.at[pl.ds(cc * _CHUNK, _CHUNK)]],
                rows_v.at[pl.ds(cc * _CHUNK, _CHUNK)],
                sem,
            )
            for cc in range(nchunk)
        ]
        for cp in copies:
            cp.wait()

        lane = lax.iota(jnp.int32, _L)

        def select(u0, carry):
            bt = u0 >> 3
            wq = u0 & 7
            offv = offs_v[pl.ds(u0 * _L, _L)]
            rowbase = u0 * _L + lane
            for rr in range(r):
                for cc in range(c):
                    kk = rr * c + cc
                    vals = plsc.load_gather(rows_v, [kk * bpw + rowbase, offv])
                    out_v[rr, bt, cc, pl.ds(wq * _L, _L)] = vals
            return carry

        lax.fori_loop(0, bpw // _L, select, jnp.int32(0))

        bt0 = wid * nbt
        for rr in range(r):
            pltpu.sync_copy(out_v.at[rr], out_hbm.at[rr, pl.ds(bt0, nbt)])

    out4 = k(projection.transpose(1, 2, 0).reshape(planes * pv, _L),
             cam_idx)
    return out4.transpose(1, 3, 0, 2).reshape(batch, r, c)


# speedup vs baseline: 26.8225x; 5.5740x over previous
import functools

import jax
import jax.numpy as jnp
from jax import lax
from jax.experimental import pallas as pl
from jax.experimental.pallas import tpu as pltpu
from jax.experimental.pallas import tpu_sc as plsc

_CHUNK = 128
_L = 16


def kernel(projection, cam_idx):
    n, r, c = projection.shape
    batch = cam_idx.shape[1]
    planes = r * c
    nmain = (n // _CHUNK) * _CHUNK
    ntail = n - nmain
    pv = nmain // _L
    info = plsc.get_sparse_core_info()
    nw = info.num_cores * info.num_subcores
    bpw = batch // nw
    nbt = bpw // _CHUNK
    ng = bpw * planes
    nchunk = ng // _CHUNK
    mesh = plsc.VectorSubcoreMesh(core_axis_name="c", subcore_axis_name="s")

    @functools.partial(
        pl.kernel,
        out_type=jax.ShapeDtypeStruct((r, batch // _CHUNK, c, _CHUNK),
                                      jnp.float32),
        mesh=mesh,
        scratch_types=[
            pltpu.VMEM((bpw,), jnp.int32),
            pltpu.VMEM((bpw,), jnp.int32),
            pltpu.VMEM((ng,), jnp.int32),
            pltpu.VMEM((ng, _L), jnp.float32),
            pltpu.VMEM((r, nbt, c, _CHUNK), jnp.float32),
            pltpu.VMEM((max(planes * ntail, _L),), jnp.float32),
            pltpu.SemaphoreType.DMA,
        ],
        compiler_params=pltpu.CompilerParams(
            use_tc_tiling_on_sc=False, needs_layout_passes=False),
    )
    def k(tab_hbm, *rest):
        if ntail:
            tail_hbm, cam_hbm, out_hbm, \
                idx_v, offs_v, gidx_v, rows_v, out_v, tail_v, sem = rest
        else:
            cam_hbm, out_hbm, \
                idx_v, offs_v, gidx_v, rows_v, out_v, tail_v, sem = rest
        wid = lax.axis_index("s") * info.num_cores + lax.axis_index("c")
        base = wid * bpw
        pltpu.sync_copy(cam_hbm.at[1, pl.ds(base, bpw)], idx_v)
        if ntail:
            pltpu.sync_copy(tail_hbm, tail_v.at[pl.ds(0, planes * ntail)])

        def prep(rr, carry):
            v = idx_v[pl.ds(rr * _L, _L)]
            offs_v[pl.ds(rr * _L, _L)] = v & (_L - 1)
            g0 = v >> 4
            if ntail:
                g0 = jnp.where(v >= nmain, 0, g0)
            for kk in range(planes):
                gidx_v[pl.ds(kk * bpw + rr * _L, _L)] = g0 + kk * pv
            return carry

        lax.fori_loop(0, bpw // _L, prep, jnp.int32(0))

        copies = [
            pltpu.async_copy(
                tab_hbm.at[gidx_v.at[pl.ds(cc * _CHUNK, _CHUNK)]],
                rows_v.at[pl.ds(cc * _CHUNK, _CHUNK)],
                sem,
            )
            for cc in range(nchunk)
        ]
        for cp in copies:
            cp.wait()

        lane = lax.iota(jnp.int32, _L)

        def select(u0, carry):
            bt = u0 >> 3
            wq = u0 & 7
            offv = offs_v[pl.ds(u0 * _L, _L)]
            rowbase = u0 * _L + lane
            if ntail:
                m = idx_v[pl.ds(u0 * _L, _L)]
                tmask = m >= nmain
                ct = jnp.where(tmask, m - nmain, 0)
            for rr in range(r):
                for cc in range(c):
                    kk = rr * c + cc
                    vals = plsc.load_gather(rows_v, [kk * bpw + rowbase, offv])
                    if ntail:
                        tvals = plsc.load_gather(tail_v, [kk * ntail + ct])
                        vals = jnp.where(tmask, tvals, vals)
                    out_v[rr, bt, cc, pl.ds(wq * _L, _L)] = vals
            return carry

        lax.fori_loop(0, bpw // _L, select, jnp.int32(0))

        bt0 = wid * nbt
        for rr in range(r):
            pltpu.sync_copy(out_v.at[rr], out_hbm.at[rr, pl.ds(bt0, nbt)])

    tab = projection.transpose(1, 2, 0)[:, :, :nmain].reshape(planes * pv, _L)
    if ntail:
        tail = projection.transpose(1, 2, 0)[:, :, nmain:].reshape(
            planes * ntail)
        out4 = k(tab, tail, cam_idx)
    else:
        out4 = k(tab, cam_idx)
    return out4.transpose(1, 3, 0, 2).reshape(batch, r, c)
